# baseline (speedup 1.0000x reference)
import functools

import jax
import jax.numpy as jnp
from jax import lax
from jax.experimental import pallas as pl
from jax.experimental.pallas import tpu as pltpu
from jax.experimental.pallas import tpu_sc as plsc

_L = 16
_NW = 32
_BW = 128


@functools.lru_cache(maxsize=None)
def _build(B, S, D):
    NG = _BW // _L
    NP = S // 2 - 1
    mesh = plsc.VectorSubcoreMesh(core_axis_name="c", subcore_axis_name="s")

    @functools.partial(
        pl.kernel,
        out_type=jax.ShapeDtypeStruct((S, D, B), jnp.float32),
        mesh=mesh,
        scratch_types=[
            pltpu.VMEM((S, _BW), jnp.int32),
            pltpu.VMEM((2, D, _BW), jnp.float32),
            pltpu.VMEM((2, D, _BW), jnp.float32),
            pltpu.VMEM((2, D, _BW), jnp.float32),
            pltpu.VMEM((2, D, _BW), jnp.float32),
            pltpu.SemaphoreType.DMA,
            pltpu.SemaphoreType.DMA,
            pltpu.SemaphoreType.DMA,
            pltpu.SemaphoreType.DMA,
            pltpu.SemaphoreType.DMA,
        ],
        compiler_params=pltpu.CompilerParams(needs_layout_passes=False),
    )
    def lor_kernel(xp, obs, out, obs_v, xb0, xb1, cb0, cb1,
                   sem_obs, sem_x0, sem_x1, sem_c0, sem_c1):
        wid = lax.axis_index("s") * 2 + lax.axis_index("c")
        b0 = wid * _BW
        xbs = (xb0, xb1)
        cbs = (cb0, cb1)
        sem_xs = (sem_x0, sem_x1)
        sem_cs = (sem_c0, sem_c1)

        def xpair_dma(p, q):
            s0 = jnp.minimum(2 * p + 1, S - 2)
            return pltpu.make_async_copy(
                xp.at[pl.ds(s0, 2), :, pl.ds(b0, _BW)], xbs[q], sem_xs[q]
            )

        def x0_dma(slot, q):
            return pltpu.make_async_copy(
                xp.at[pl.ds(0, 1), :, pl.ds(b0, _BW)],
                cbs[q].at[pl.ds(slot, 1)], sem_cs[q]
            )

        def out_dma(q, t0):
            return pltpu.make_async_copy(
                cbs[q], out.at[pl.ds(t0, 2), :, pl.ds(b0, _BW)], sem_cs[q]
            )

        obs_cp = pltpu.make_async_copy(
            obs.at[:, pl.ds(b0, _BW)], obs_v, sem_obs
        )
        obs_cp.start()
        x0_dma(0, 1).start()
        x0_dma(1, 1).start()
        xpair_dma(0, 0).start()
        x0_dma(0, 1).wait()
        x0_dma(1, 1).wait()
        out_dma(1, 0).start()
        obs_cp.wait()

        def do_pair(p, q, first=False):
            t0 = 2 * p + 2
            xpair_dma(p + 1, 1 - q).start()
            xpair_dma(p, q).wait()
            if not first:
                out_dma(q, t0 - 4).wait()
            xsrc = xbs[q]
            cprev = cbs[1 - q]
            cdst = cbs[q]
            m0 = [obs_v[t0 - 1, pl.ds(k * _L, _L)] != 0 for k in range(NG)]
            m1 = [obs_v[t0, pl.ds(k * _L, _L)] != 0 for k in range(NG)]

            @plsc.parallel_loop(0, D, step=1, unroll=16)
            def _upd0(d):
                for k in range(NG):
                    sl = pl.ds(k * _L, _L)
                    cdst[0, d, sl] = jnp.where(
                        m0[k], xsrc[0, d, sl], cprev[1, d, sl]
                    )

            @plsc.parallel_loop(0, D, step=1, unroll=16)
            def _upd1(d):
                for k in range(NG):
                    sl = pl.ds(k * _L, _L)
                    cdst[1, d, sl] = jnp.where(
                        m1[k], xsrc[1, d, sl], cdst[0, d, sl]
                    )

            out_dma(q, t0).start()

        def body(i, acc):
            do_pair(2 * i + 1, 1)
            do_pair(2 * i + 2, 0)
            return acc

        do_pair(0, 0, first=True)
        lax.fori_loop(0, (NP - 1) // 2, body, 0)

        out_dma((NP - 2) % 2, S - 4).wait()
        out_dma((NP - 1) % 2, S - 2).wait()
        xpair_dma(NP, NP % 2).wait()

    return lor_kernel


def kernel(x, observed):
    B, S, D = x.shape
    xp = jnp.transpose(x, (1, 2, 0))
    obsT = jnp.transpose(observed.astype(jnp.int32), (1, 0))
    outp = _build(B, S, D)(xp, obsT)
    return jnp.transpose(outp, (2, 0, 1))


# speedup vs baseline: 1.8973x; 1.8973x over previous
import functools

import jax
import jax.numpy as jnp
from jax import lax
from jax.experimental import pallas as pl
from jax.experimental.pallas import tpu as pltpu
from jax.experimental.pallas import tpu_sc as plsc

_L = 16
_NW = 32
_BW = 128


@functools.lru_cache(maxsize=None)
def _build(B, S, D):
    NG = _BW // _L
    NP = S // 2 - 1
    mesh = plsc.VectorSubcoreMesh(core_axis_name="c", subcore_axis_name="s")

    @functools.partial(
        pl.kernel,
        out_type=jax.ShapeDtypeStruct((S, D, B), jnp.float32),
        mesh=mesh,
        scratch_types=[
            pltpu.VMEM((S, _BW), jnp.int32),
            pltpu.VMEM((2, D, _BW), jnp.float32),
            pltpu.VMEM((2, D, _BW), jnp.float32),
            pltpu.VMEM((2, D, _BW), jnp.float32),
            pltpu.VMEM((2, D, _BW), jnp.float32),
            pltpu.SemaphoreType.DMA,
            pltpu.SemaphoreType.DMA,
            pltpu.SemaphoreType.DMA,
            pltpu.SemaphoreType.DMA,
            pltpu.SemaphoreType.DMA,
        ],
        compiler_params=pltpu.CompilerParams(needs_layout_passes=False),
    )
    def lor_kernel(xp, obs, out, obs_v, xb0, xb1, cb0, cb1,
                   sem_obs, sem_x0, sem_x1, sem_c0, sem_c1):
        wid = lax.axis_index("s") * 2 + lax.axis_index("c")
        b0 = wid * _BW
        xbs = (xb0, xb1)
        cbs = (cb0, cb1)
        sem_xs = (sem_x0, sem_x1)
        sem_cs = (sem_c0, sem_c1)

        def xpair_dma(p, q):
            s0 = jnp.minimum(2 * p + 1, S - 2)
            return pltpu.make_async_copy(
                xp.at[pl.ds(s0, 2), :, pl.ds(b0, _BW)], xbs[q], sem_xs[q]
            )

        def x0_dma(slot, q):
            return pltpu.make_async_copy(
                xp.at[pl.ds(0, 1), :, pl.ds(b0, _BW)],
                cbs[q].at[pl.ds(slot, 1)], sem_cs[q]
            )

        def out_dma(q, t0):
            return pltpu.make_async_copy(
                cbs[q], out.at[pl.ds(t0, 2), :, pl.ds(b0, _BW)], sem_cs[q]
            )

        obs_cp = pltpu.make_async_copy(
            obs.at[:, pl.ds(b0, _BW)], obs_v, sem_obs
        )
        obs_cp.start()
        x0_dma(0, 1).start()
        x0_dma(1, 1).start()
        xpair_dma(0, 0).start()
        x0_dma(0, 1).wait()
        x0_dma(1, 1).wait()
        out_dma(1, 0).start()
        obs_cp.wait()

        def do_pair(p, q, first=False):
            t0 = 2 * p + 2
            xpair_dma(p + 1, 1 - q).start()
            xpair_dma(p, q).wait()
            if not first:
                out_dma(q, t0 - 4).wait()
            xsrc = xbs[q]
            cprev = cbs[1 - q]
            cdst = cbs[q]
            m0 = [obs_v[t0 - 1, pl.ds(k * _L, _L)] != 0 for k in range(NG)]
            m1 = [obs_v[t0, pl.ds(k * _L, _L)] != 0 for k in range(NG)]

            @plsc.parallel_loop(0, D, step=1, unroll=8)
            def _upd0(d):
                for k in range(NG):
                    sl = pl.ds(k * _L, _L)
                    cdst[0, d, sl] = jnp.where(
                        m0[k], xsrc[0, d, sl], cprev[1, d, sl]
                    )

            @plsc.parallel_loop(0, D, step=1, unroll=8)
            def _upd1(d):
                for k in range(NG):
                    sl = pl.ds(k * _L, _L)
                    cdst[1, d, sl] = jnp.where(
                        m1[k], xsrc[1, d, sl], cdst[0, d, sl]
                    )

            out_dma(q, t0).start()

        def body(i, acc):
            do_pair(2 * i + 1, 1)
            do_pair(2 * i + 2, 0)
            return acc

        do_pair(0, 0, first=True)
        lax.fori_loop(0, (NP - 1) // 2, body, 0)

        out_dma((NP - 2) % 2, S - 4).wait()
        out_dma((NP - 1) % 2, S - 2).wait()
        xpair_dma(NP, NP % 2).wait()

    return lor_kernel


def kernel(x, observed):
    B, S, D = x.shape
    xp = jnp.transpose(x, (1, 2, 0))
    obsT = jnp.transpose(observed.astype(jnp.int32), (1, 0))
    outp = _build(B, S, D)(xp, obsT)
    return jnp.transpose(outp, (2, 0, 1))
